# baseline (speedup 1.0000x reference)
import functools

import jax
import jax.numpy as jnp
from jax import lax
from jax.experimental import pallas as pl
from jax.experimental.pallas import tpu as pltpu
from jax.experimental.pallas import tpu_sc as plsc

N = 10000
E = 320000
D = 128
NC = 2
NS = 16
NW = NC * NS
EPW = E // NW
B = 80
C = EPW // B
RPT = N // NS
KD = 25
KC = 5

_mesh = plsc.VectorSubcoreMesh(core_axis_name="c", subcore_axis_name="s")
_sc_params = pltpu.CompilerParams(use_tc_tiling_on_sc=False)


@functools.partial(
    pl.kernel,
    out_type=jax.ShapeDtypeStruct((NC, NS, RPT, 32), jnp.float32),
    mesh=_mesh,
    compiler_params=_sc_params,
    scratch_types=[
        pltpu.VMEM((C, B), jnp.int32),
        pltpu.VMEM((B, 32), jnp.float32),
        pltpu.VMEM_SHARED((N, 32), jnp.float32),
        pltpu.SemaphoreType.DMA,
        pltpu.SemaphoreType.DMA,
    ],
)
def _deg_kernel(row_hbm, zeros_hbm, ones_hbm, out_hbm, idx_v, ones_v, acc_sh,
                sd0, sd1):
    c = lax.axis_index("c")
    s = lax.axis_index("s")
    wid = c * NS + s
    pltpu.sync_copy(zeros_hbm, acc_sh.at[pl.ds(s * RPT, RPT)])
    pltpu.sync_copy(row_hbm.at[wid], idx_v)
    pltpu.sync_copy(ones_hbm, ones_v)
    plsc.subcore_barrier()

    sd = (sd0, sd1)
    TD = C // KD

    def issue_d(t):
        for k in range(KD):
            pltpu.async_copy(ones_v, acc_sh.at[idx_v.at[t * KD + k]],
                             sd[t % 2], add=True)

    def drain_d(t):
        for k in range(KD):
            pltpu.make_async_copy(ones_v, acc_sh.at[idx_v.at[t * KD + k]],
                                  sd[t % 2]).wait()

    issue_d(0)
    for t in range(1, TD):
        issue_d(t)
        drain_d(t - 1)
    drain_d(TD - 1)
    plsc.subcore_barrier()
    pltpu.sync_copy(acc_sh.at[pl.ds(s * RPT, RPT)], out_hbm.at[c, s])


def _make_conv_kernel(Dr):
    @functools.partial(
        pl.kernel,
        out_type=jax.ShapeDtypeStruct((NC, NS, RPT, Dr), jnp.float32),
        mesh=_mesh,
        compiler_params=_sc_params,
        scratch_types=[
            pltpu.VMEM((C, B), jnp.int32),
            pltpu.VMEM((C, B), jnp.int32),
            pltpu.VMEM((3, KC, B, Dr), jnp.float32),
            pltpu.VMEM_SHARED((N, Dr), jnp.float32),
            pltpu.SemaphoreType.DMA,
            pltpu.SemaphoreType.DMA,
            pltpu.SemaphoreType.DMA,
            pltpu.SemaphoreType.DMA,
            pltpu.SemaphoreType.DMA,
            pltpu.SemaphoreType.DMA,
        ],
    )
    def _conv_kernel(table_hbm, row_hbm, col_hbm, zeros_hbm, out_hbm,
                     row_v, col_v, buf_v, acc_sh,
                     sg0, sg1, sg2, ss0, ss1, ss2):
        c = lax.axis_index("c")
        s = lax.axis_index("s")
        wid = c * NS + s
        sg = (sg0, sg1, sg2)
        ss = (ss0, ss1, ss2)

        pltpu.sync_copy(zeros_hbm, acc_sh.at[pl.ds(s * RPT, RPT)])
        pltpu.sync_copy(row_hbm.at[wid], row_v)
        pltpu.sync_copy(col_hbm.at[wid], col_v)
        plsc.subcore_barrier()

        T = C // KC

        def issue_g(t, m):
            for k in range(KC):
                pltpu.async_copy(table_hbm.at[row_v.at[t * KC + k]],
                                 buf_v.at[m, k], sg[m])

        def drain_g(t, m):
            for k in range(KC):
                pltpu.make_async_copy(table_hbm.at[row_v.at[t * KC + k]],
                                      buf_v.at[m, k], sg[m]).wait()

        def issue_s(t, m):
            for k in range(KC):
                pltpu.async_copy(buf_v.at[m, k],
                                 acc_sh.at[col_v.at[t * KC + k]],
                                 ss[m], add=True)

        def drain_s(t, m):
            for k in range(KC):
                pltpu.make_async_copy(buf_v.at[m, k],
                                      acc_sh.at[col_v.at[t * KC + k]],
                                      ss[m]).wait()

        issue_g(0, 0)
        issue_g(1, 1)
        drain_g(0, 0)
        issue_s(0, 0)
        issue_g(2, 2)

        def body(i, _):
            t = 3 * i + 1
            for dm in range(3):
                m = (1 + dm) % 3
                drain_g(t + dm, m)
                issue_s(t + dm, m)
                drain_s(t + dm - 1, (m + 2) % 3)
                issue_g(t + dm + 2, (m + 2) % 3)
            return _

        lax.fori_loop(0, (T - 4) // 3, body, None)
        for t in (T - 3, T - 2, T - 1):
            m = t % 3
            drain_g(t, m)
            issue_s(t, m)
            drain_s(t - 1, (m + 2) % 3)
            if t + 2 < T:
                issue_g(t + 2, (m + 2) % 3)
        drain_s(T - 1, (T - 1) % 3)

        plsc.subcore_barrier()
        pltpu.sync_copy(acc_sh.at[pl.ds(s * RPT, RPT)], out_hbm.at[c, s])

    return _conv_kernel


_conv32 = _make_conv_kernel(32)
_conv8 = _make_conv_kernel(8)


def _iota2(shape, dim):
    return lax.broadcasted_iota(jnp.int32, shape, dim)


def _dis32(d0_ref, d1_ref):
    i = _iota2((128, 128), 0)
    j = _iota2((128, 128), 1)
    r32 = ((i % 32 == 0) & (j // 32 == i // 32)).astype(jnp.float32)
    dsum = jnp.dot(d0_ref[...] + d1_ref[...], r32,
                   preferred_element_type=jnp.float32)
    return lax.rsqrt(dsum + 1.0)


def _tc1_body(xp_ref, bd1_ref, b1p_ref, d0_ref, d1_ref, ht_ref):
    h = jnp.dot(xp_ref[...], bd1_ref[...],
                preferred_element_type=jnp.float32) + b1p_ref[...]
    ht_ref[...] = h * _dis32(d0_ref, d1_ref)


def _tc2_body(p0_ref, p1_ref, ht1_ref, d0_ref, d1_ref, bd2_ref, b2q_ref,
              ht2_ref, dis8_ref):
    dis32 = _dis32(d0_ref, d1_ref)
    s = p0_ref[...] + p1_ref[...] + ht1_ref[...]
    out1 = jnp.maximum(dis32 * s, 0.0)
    h2 = jnp.dot(out1, bd2_ref[...],
                 preferred_element_type=jnp.float32) + b2q_ref[...]
    i = _iota2((128, 32), 0)
    j = _iota2((128, 32), 1)
    s8 = (i == 32 * (j // 8)).astype(jnp.float32)
    dis8 = jnp.dot(dis32, s8, preferred_element_type=jnp.float32)
    ht2_ref[...] = h2 * dis8
    dis8_ref[...] = dis8


def _tc3_body(q0_ref, q1_ref, ht2_ref, dis8_ref, x1p_ref, wl_ref, bl_ref,
              z_ref, emb_ref):
    s = q0_ref[...] + q1_ref[...] + ht2_ref[...]
    out2 = dis8_ref[...] * s
    m = jnp.max(out2, axis=1, keepdims=True)
    e = jnp.exp(out2 - m)
    i = _iota2((128, 128), 0)
    j = _iota2((128, 128), 1)
    g8 = ((i // 8) == (j // 8)).astype(jnp.float32)
    ssum = jnp.dot(e, g8, preferred_element_type=jnp.float32)
    emb = (out2 - m) - jnp.log(ssum)
    wl = wl_ref[...]
    it = _iota2((8, 128), 0)
    jt = _iota2((8, 128), 1)
    tile8 = (jt % 8 == it).astype(jnp.float32)
    wlp = jnp.dot(wl[:, 0:8], tile8, preferred_element_type=jnp.float32)
    ig = _iota2((128, 16), 0)
    jg = _iota2((128, 16), 1)
    gsel = ((ig // 8) == jg).astype(jnp.float32)
    zq = jnp.dot(emb * wlp, gsel, preferred_element_type=jnp.float32)
    z = zq + x1p_ref[...] * wl[:, 8:9] + bl_ref[...]
    z_ref[...] = jnp.maximum(z, 0.0)
    emb_ref[...] = emb


def kernel(x, edge_index, x1, W1, b1, W2, b2, Wl, bl):
    f32 = jnp.float32
    row3 = edge_index[0].reshape(NW, C, B)
    col3 = edge_index[1].reshape(NW, C, B)
    zeros32 = jnp.zeros((RPT, 32), f32)
    zeros8 = jnp.zeros((RPT, 8), f32)
    ones_hbm = jnp.zeros((B, 32), f32).at[:, 0].set(1.0)

    degp = _deg_kernel(row3, zeros32, ones_hbm)
    degp3 = degp.reshape(NC, N // 4, 128)
    d0p, d1p = degp3[0], degp3[1]

    bd1 = jax.scipy.linalg.block_diag(*([W1.T] * 4))
    bd2 = jax.scipy.linalg.block_diag(*([W2.T] * 4))
    b1p = jnp.tile(b1, 4).reshape(1, 128)
    b2q = jnp.tile(b2, 4).reshape(1, 32)

    ht1p = pl.pallas_call(
        _tc1_body,
        out_shape=jax.ShapeDtypeStruct((N // 4, 128), f32),
    )(x.reshape(N // 4, 512), bd1, b1p, d0p, d1p)

    p3 = _conv32(ht1p.reshape(N, 32), row3, col3,
                 zeros32).reshape(NC, N // 4, 128)

    ht2q, dis8q = pl.pallas_call(
        _tc2_body,
        out_shape=[jax.ShapeDtypeStruct((N // 4, 32), f32),
                   jax.ShapeDtypeStruct((N // 4, 32), f32)],
    )(p3[0], p3[1], ht1p, d0p, d1p, bd2, b2q)

    ht2lin = ht2q.reshape(N, 8)
    q3 = _conv8(ht2lin, row3, col3, zeros8).reshape(NC, N // 16, 128)

    z16, embp = pl.pallas_call(
        _tc3_body,
        out_shape=[jax.ShapeDtypeStruct((N // 16, 16), f32),
                   jax.ShapeDtypeStruct((N // 16, 128), f32)],
    )(q3[0], q3[1],
      ht2q.reshape(N // 16, 128), dis8q.reshape(N // 16, 128),
      x1.reshape(N // 16, 16), Wl, bl.reshape(1, 1))

    return (z16.reshape(N, 1), embp.reshape(N, 8))


# speedup vs baseline: 1.0017x; 1.0017x over previous
import functools

import jax
import jax.numpy as jnp
from jax import lax
from jax.experimental import pallas as pl
from jax.experimental.pallas import tpu as pltpu
from jax.experimental.pallas import tpu_sc as plsc

N = 10000
E = 320000
D = 128
NC = 2
NS = 16
NW = NC * NS
EPW = E // NW
B = 80
C = EPW // B
RPT = N // NS
KD = 25

_mesh = plsc.VectorSubcoreMesh(core_axis_name="c", subcore_axis_name="s")
_sc_params = pltpu.CompilerParams(use_tc_tiling_on_sc=False)


@functools.partial(
    pl.kernel,
    out_type=jax.ShapeDtypeStruct((NC, NS, RPT, 32), jnp.float32),
    mesh=_mesh,
    compiler_params=_sc_params,
    scratch_types=[
        pltpu.VMEM((C, B), jnp.int32),
        pltpu.VMEM((B, 32), jnp.float32),
        pltpu.VMEM_SHARED((N, 32), jnp.float32),
        pltpu.SemaphoreType.DMA,
        pltpu.SemaphoreType.DMA,
    ],
)
def _deg_kernel(row_hbm, zeros_hbm, ones_hbm, out_hbm, idx_v, ones_v, acc_sh,
                sd0, sd1):
    c = lax.axis_index("c")
    s = lax.axis_index("s")
    wid = c * NS + s
    pltpu.sync_copy(zeros_hbm, acc_sh.at[pl.ds(s * RPT, RPT)])
    pltpu.sync_copy(row_hbm.at[wid], idx_v)
    pltpu.sync_copy(ones_hbm, ones_v)
    plsc.subcore_barrier()

    sd = (sd0, sd1)
    TD = C // KD

    def issue_d(t):
        for k in range(KD):
            pltpu.async_copy(ones_v, acc_sh.at[idx_v.at[t * KD + k]],
                             sd[t % 2], add=True)

    def drain_d(t):
        for k in range(KD):
            pltpu.make_async_copy(ones_v, acc_sh.at[idx_v.at[t * KD + k]],
                                  sd[t % 2]).wait()

    issue_d(0)
    for t in range(1, TD):
        issue_d(t)
        drain_d(t - 1)
    drain_d(TD - 1)
    plsc.subcore_barrier()
    pltpu.sync_copy(acc_sh.at[pl.ds(s * RPT, RPT)], out_hbm.at[c, s])


def _make_conv_kernel(Dr, KC):
    @functools.partial(
        pl.kernel,
        out_type=jax.ShapeDtypeStruct((NC, NS, RPT, Dr), jnp.float32),
        mesh=_mesh,
        compiler_params=_sc_params,
        scratch_types=[
            pltpu.VMEM((C, B), jnp.int32),
            pltpu.VMEM((C, B), jnp.int32),
            pltpu.VMEM((3, KC, B, Dr), jnp.float32),
            pltpu.VMEM_SHARED((N, Dr), jnp.float32),
            pltpu.SemaphoreType.DMA,
            pltpu.SemaphoreType.DMA,
            pltpu.SemaphoreType.DMA,
            pltpu.SemaphoreType.DMA,
            pltpu.SemaphoreType.DMA,
            pltpu.SemaphoreType.DMA,
        ],
    )
    def _conv_kernel(table_hbm, row_hbm, col_hbm, zeros_hbm, out_hbm,
                     row_v, col_v, buf_v, acc_sh,
                     sg0, sg1, sg2, ss0, ss1, ss2):
        c = lax.axis_index("c")
        s = lax.axis_index("s")
        wid = c * NS + s
        sg = (sg0, sg1, sg2)
        ss = (ss0, ss1, ss2)

        pltpu.sync_copy(zeros_hbm, acc_sh.at[pl.ds(s * RPT, RPT)])
        pltpu.sync_copy(row_hbm.at[wid], row_v)
        pltpu.sync_copy(col_hbm.at[wid], col_v)
        plsc.subcore_barrier()

        T = C // KC

        def issue_g(t, m):
            for k in range(KC):
                pltpu.async_copy(table_hbm.at[row_v.at[t * KC + k]],
                                 buf_v.at[m, k], sg[m])

        def drain_g(t, m):
            for k in range(KC):
                pltpu.make_async_copy(table_hbm.at[row_v.at[t * KC + k]],
                                      buf_v.at[m, k], sg[m]).wait()

        def issue_s(t, m):
            for k in range(KC):
                pltpu.async_copy(buf_v.at[m, k],
                                 acc_sh.at[col_v.at[t * KC + k]],
                                 ss[m], add=True)

        def drain_s(t, m):
            for k in range(KC):
                pltpu.make_async_copy(buf_v.at[m, k],
                                      acc_sh.at[col_v.at[t * KC + k]],
                                      ss[m]).wait()

        issue_g(0, 0)
        issue_g(1, 1)
        if T >= 7:
            drain_g(0, 0)
            issue_s(0, 0)
            issue_g(2, 2)

            def body(i, _):
                t = 3 * i + 1
                for dm in range(3):
                    m = (1 + dm) % 3
                    drain_g(t + dm, m)
                    issue_s(t + dm, m)
                    drain_s(t + dm - 1, (m + 2) % 3)
                    issue_g(t + dm + 2, (m + 2) % 3)
                return _

            assert (T - 4) % 3 == 0
            lax.fori_loop(0, (T - 4) // 3, body, None)
            tail = (T - 3, T - 2, T - 1)
        else:
            tail = range(T)
        for t in tail:
            m = t % 3
            drain_g(t, m)
            issue_s(t, m)
            if t >= 1:
                drain_s(t - 1, (t - 1) % 3)
            if t + 2 < T:
                issue_g(t + 2, (t + 2) % 3)
        drain_s(T - 1, (T - 1) % 3)

        plsc.subcore_barrier()
        pltpu.sync_copy(acc_sh.at[pl.ds(s * RPT, RPT)], out_hbm.at[c, s])

    return _conv_kernel


_conv32 = _make_conv_kernel(32, 5)
_conv8 = _make_conv_kernel(8, 25)


def _iota2(shape, dim):
    return lax.broadcasted_iota(jnp.int32, shape, dim)


def _dis32(d0_ref, d1_ref):
    i = _iota2((128, 128), 0)
    j = _iota2((128, 128), 1)
    r32 = ((i % 32 == 0) & (j // 32 == i // 32)).astype(jnp.float32)
    dsum = jnp.dot(d0_ref[...] + d1_ref[...], r32,
                   preferred_element_type=jnp.float32)
    return lax.rsqrt(dsum + 1.0)


def _tc1_body(xp_ref, bd1_ref, b1p_ref, d0_ref, d1_ref, ht_ref):
    h = jnp.dot(xp_ref[...], bd1_ref[...],
                preferred_element_type=jnp.float32) + b1p_ref[...]
    ht_ref[...] = h * _dis32(d0_ref, d1_ref)


def _tc2_body(p0_ref, p1_ref, ht1_ref, d0_ref, d1_ref, bd2_ref, b2q_ref,
              ht2_ref, dis8_ref):
    dis32 = _dis32(d0_ref, d1_ref)
    s = p0_ref[...] + p1_ref[...] + ht1_ref[...]
    out1 = jnp.maximum(dis32 * s, 0.0)
    h2 = jnp.dot(out1, bd2_ref[...],
                 preferred_element_type=jnp.float32) + b2q_ref[...]
    i = _iota2((128, 32), 0)
    j = _iota2((128, 32), 1)
    s8 = (i == 32 * (j // 8)).astype(jnp.float32)
    dis8 = jnp.dot(dis32, s8, preferred_element_type=jnp.float32)
    ht2_ref[...] = h2 * dis8
    dis8_ref[...] = dis8


def _tc3_body(q0_ref, q1_ref, ht2_ref, dis8_ref, x1p_ref, wl_ref, bl_ref,
              z_ref, emb_ref):
    s = q0_ref[...] + q1_ref[...] + ht2_ref[...]
    out2 = dis8_ref[...] * s
    m = jnp.max(out2, axis=1, keepdims=True)
    e = jnp.exp(out2 - m)
    i = _iota2((128, 128), 0)
    j = _iota2((128, 128), 1)
    g8 = ((i // 8) == (j // 8)).astype(jnp.float32)
    ssum = jnp.dot(e, g8, preferred_element_type=jnp.float32)
    emb = (out2 - m) - jnp.log(ssum)
    wl = wl_ref[...]
    it = _iota2((8, 128), 0)
    jt = _iota2((8, 128), 1)
    tile8 = (jt % 8 == it).astype(jnp.float32)
    wlp = jnp.dot(wl[:, 0:8], tile8, preferred_element_type=jnp.float32)
    ig = _iota2((128, 16), 0)
    jg = _iota2((128, 16), 1)
    gsel = ((ig // 8) == jg).astype(jnp.float32)
    zq = jnp.dot(emb * wlp, gsel, preferred_element_type=jnp.float32)
    z = zq + x1p_ref[...] * wl[:, 8:9] + bl_ref[...]
    z_ref[...] = jnp.maximum(z, 0.0)
    emb_ref[...] = emb


def kernel(x, edge_index, x1, W1, b1, W2, b2, Wl, bl):
    f32 = jnp.float32
    row3 = edge_index[0].reshape(NW, C, B)
    col3 = edge_index[1].reshape(NW, C, B)
    zeros32 = jnp.zeros((RPT, 32), f32)
    zeros8 = jnp.zeros((RPT, 8), f32)
    ones_hbm = jnp.zeros((B, 32), f32).at[:, 0].set(1.0)

    degp = _deg_kernel(row3, zeros32, ones_hbm)
    degp3 = degp.reshape(NC, N // 4, 128)
    d0p, d1p = degp3[0], degp3[1]

    bd1 = jax.scipy.linalg.block_diag(*([W1.T] * 4))
    bd2 = jax.scipy.linalg.block_diag(*([W2.T] * 4))
    b1p = jnp.tile(b1, 4).reshape(1, 128)
    b2q = jnp.tile(b2, 4).reshape(1, 32)

    ht1p = pl.pallas_call(
        _tc1_body,
        out_shape=jax.ShapeDtypeStruct((N // 4, 128), f32),
    )(x.reshape(N // 4, 512), bd1, b1p, d0p, d1p)

    p3 = _conv32(ht1p.reshape(N, 32), row3, col3,
                 zeros32).reshape(NC, N // 4, 128)

    ht2q, dis8q = pl.pallas_call(
        _tc2_body,
        out_shape=[jax.ShapeDtypeStruct((N // 4, 32), f32),
                   jax.ShapeDtypeStruct((N // 4, 32), f32)],
    )(p3[0], p3[1], ht1p, d0p, d1p, bd2, b2q)

    ht2lin = ht2q.reshape(N, 8)
    q3 = _conv8(ht2lin, row3, col3, zeros8).reshape(NC, N // 16, 128)

    z16, embp = pl.pallas_call(
        _tc3_body,
        out_shape=[jax.ShapeDtypeStruct((N // 16, 16), f32),
                   jax.ShapeDtypeStruct((N // 16, 128), f32)],
    )(q3[0], q3[1],
      ht2q.reshape(N // 16, 128), dis8q.reshape(N // 16, 128),
      x1.reshape(N // 16, 16), Wl, bl.reshape(1, 1))

    return (z16.reshape(N, 1), embp.reshape(N, 8))
